# baseline (speedup 1.0000x reference)
import jax
import jax.numpy as jnp
from jax import lax
from jax.experimental import pallas as pl
from jax.experimental.pallas import tpu as pltpu
from jax.experimental.pallas import tpu_sc as plsc

B, S, HIDDEN = 4, 2048, 1024
LANES = 16
NUM_CORES = 2
NUM_SUBCORES = 16
NW = NUM_CORES * NUM_SUBCORES
TOKENS = B * S
PER_W = TOKENS // NW
W_PER_ROW = S // PER_W
CHUNK = 16
NCHUNK = PER_W // CHUNK
CPH = HIDDEN // LANES
NG = 2
NO = 3


def _embed_body(ids, pids, wtab, ptab, out, idx_w, idx_p, buf_w, buf_p, buf_o,
                sem_w, sem_p, sem_o):
    wid = lax.axis_index("s") * NUM_CORES + lax.axis_index("c")
    row = wid // W_PER_ROW
    col = (wid % W_PER_ROW) * PER_W
    pltpu.sync_copy(ids.at[row, pl.ds(col, PER_W)], idx_w)
    pltpu.sync_copy(pids.at[row, pl.ds(col, PER_W)], idx_p)

    def idx_slice(ref, j):
        return ref.at[pl.ds(pl.multiple_of(j * CHUNK, 8), CHUNK)]

    H = CHUNK // 2

    def half_slice(ref, j, h):
        return ref.at[pl.ds(pl.multiple_of(j * CHUNK + h * H, 8), H)]

    def gathers(j, s2):
        cs = []
        for h in range(2):
            cs.append(pltpu.make_async_copy(
                wtab.at[half_slice(idx_w, j, h)],
                buf_w.at[s2, pl.ds(h * H, H)], sem_w.at[s2]))
            cs.append(pltpu.make_async_copy(
                ptab.at[half_slice(idx_p, j, h)],
                buf_p.at[s2, pl.ds(h * H, H)], sem_p.at[s2]))
        return cs

    def issue_gathers(j, s2):
        for c in gathers(j, s2):
            c.start()

    def wait_gathers(j, s2):
        for c in gathers(j, s2):
            c.wait()

    def add_chunk(s2, s3):
        @plsc.parallel_loop(0, CHUNK * CPH, unroll=16)
        def _add(t):
            r = t >> 6
            c = pl.multiple_of((t & (CPH - 1)) << 4, LANES)
            buf_o[s3, r, pl.ds(c, LANES)] = (buf_w[s2, r, pl.ds(c, LANES)]
                                             + buf_p[s2, r, pl.ds(c, LANES)])

    def out_copy(j, s3):
        tok = pl.multiple_of(col + j * CHUNK, 8)
        return pltpu.make_async_copy(buf_o.at[s3], out.at[row, pl.ds(tok, CHUNK)],
                                     sem_o.at[s3])

    issue_gathers(0, 0)
    issue_gathers(1, 1)
    for j in range(3):
        s2, s3 = j % NG, j % NO
        wait_gathers(j, s2)
        add_chunk(s2, s3)
        issue_gathers(j + NG, s2)
        out_copy(j, s3).start()

    def steady(j, carry):
        s2 = lax.rem(j, NG)
        s3 = lax.rem(j, NO)
        wait_gathers(j, s2)
        out_copy(j - NO, s3).wait()
        add_chunk(s2, s3)
        issue_gathers(j + NG, s2)
        out_copy(j, s3).start()
        return carry

    lax.fori_loop(3, NCHUNK - NG, steady, 0)

    for j in range(NCHUNK - NG, NCHUNK):
        s2, s3 = j % NG, j % NO
        wait_gathers(j, s2)
        out_copy(j - NO, s3).wait()
        add_chunk(s2, s3)
        out_copy(j, s3).start()
    for j in range(NCHUNK - NO, NCHUNK):
        out_copy(j, j % NO).wait()


@jax.jit
def kernel(input_ids, position_ids, attention_mask, word_emb, pos_emb):
    ids = input_ids.astype(jnp.int32)
    pids = position_ids.astype(jnp.int32)
    mesh = plsc.VectorSubcoreMesh(
        core_axis_name="c",
        subcore_axis_name="s",
        num_cores=NUM_CORES,
        num_subcores=NUM_SUBCORES,
    )
    out = pl.kernel(
        _embed_body,
        out_type=jax.ShapeDtypeStruct((B, S, HIDDEN), jnp.float32),
        mesh=mesh,
        scratch_types=[
            pltpu.VMEM((PER_W,), jnp.int32),
            pltpu.VMEM((PER_W,), jnp.int32),
            pltpu.VMEM((NG, CHUNK, HIDDEN), jnp.float32),
            pltpu.VMEM((NG, CHUNK, HIDDEN), jnp.float32),
            pltpu.VMEM((NO, CHUNK, HIDDEN), jnp.float32),
            pltpu.SemaphoreType.DMA((NG,)),
            pltpu.SemaphoreType.DMA((NG,)),
            pltpu.SemaphoreType.DMA((NO,)),
        ],
    )(ids, pids, word_emb, pos_emb)
    return out, attention_mask


# speedup vs baseline: 1.0045x; 1.0045x over previous
import jax
import jax.numpy as jnp
from jax import lax
from jax.experimental import pallas as pl
from jax.experimental.pallas import tpu as pltpu
from jax.experimental.pallas import tpu_sc as plsc

B, S, HIDDEN = 4, 2048, 1024
LANES = 16
NUM_CORES = 2
NUM_SUBCORES = 16
NW = NUM_CORES * NUM_SUBCORES
TOKENS = B * S
PER_W = TOKENS // NW
W_PER_ROW = S // PER_W
CHUNK = 16
NCHUNK = PER_W // CHUNK
CPH = HIDDEN // LANES
NG = 2
NO = 3


def _embed_body(ids, pids, wtab, ptab, out, idx_w, idx_p, buf_w, buf_p, buf_o,
                sem_w, sem_p, sem_o):
    wid = lax.axis_index("s") * NUM_CORES + lax.axis_index("c")
    row = wid // W_PER_ROW
    col = (wid % W_PER_ROW) * PER_W
    pltpu.sync_copy(ids.at[row, pl.ds(col, PER_W)], idx_w)
    pltpu.sync_copy(pids.at[row, pl.ds(col, PER_W)], idx_p)

    def idx_slice(ref, j):
        return ref.at[pl.ds(pl.multiple_of(j * CHUNK, 8), CHUNK)]

    def gathers(j, s2):
        off = pl.multiple_of(j * CHUNK, LANES)
        iw = idx_w[pl.ds(off, LANES)]
        ip = idx_p[pl.ds(off, LANES)]
        cw = pltpu.make_async_copy(wtab.at[iw], buf_w.at[s2], sem_w.at[s2])
        cp = pltpu.make_async_copy(ptab.at[ip], buf_p.at[s2], sem_p.at[s2])
        return cw, cp

    def issue_gathers(j, s2):
        cw, cp = gathers(j, s2)
        cw.start()
        cp.start()

    def wait_gathers(j, s2):
        cw, cp = gathers(j, s2)
        cw.wait()
        cp.wait()

    def add_chunk(s2, s3):
        @plsc.parallel_loop(0, CHUNK * CPH, unroll=16)
        def _add(t):
            r = t >> 6
            c = pl.multiple_of((t & (CPH - 1)) << 4, LANES)
            buf_o[s3, r, pl.ds(c, LANES)] = (buf_w[s2, r, pl.ds(c, LANES)]
                                             + buf_p[s2, r, pl.ds(c, LANES)])

    def out_copy(j, s3):
        tok = pl.multiple_of(col + j * CHUNK, 8)
        return pltpu.make_async_copy(buf_o.at[s3], out.at[row, pl.ds(tok, CHUNK)],
                                     sem_o.at[s3])

    issue_gathers(0, 0)
    issue_gathers(1, 1)
    for j in range(3):
        s2, s3 = j % NG, j % NO
        wait_gathers(j, s2)
        add_chunk(s2, s3)
        issue_gathers(j + NG, s2)
        out_copy(j, s3).start()

    def steady(j, carry):
        s2 = lax.rem(j, NG)
        s3 = lax.rem(j, NO)
        wait_gathers(j, s2)
        out_copy(j - NO, s3).wait()
        add_chunk(s2, s3)
        issue_gathers(j + NG, s2)
        out_copy(j, s3).start()
        return carry

    lax.fori_loop(3, NCHUNK - NG, steady, 0)

    for j in range(NCHUNK - NG, NCHUNK):
        s2, s3 = j % NG, j % NO
        wait_gathers(j, s2)
        out_copy(j - NO, s3).wait()
        add_chunk(s2, s3)
        out_copy(j, s3).start()
    for j in range(NCHUNK - NO, NCHUNK):
        out_copy(j, j % NO).wait()


@jax.jit
def kernel(input_ids, position_ids, attention_mask, word_emb, pos_emb):
    ids = input_ids.astype(jnp.int32)
    pids = position_ids.astype(jnp.int32)
    mesh = plsc.VectorSubcoreMesh(
        core_axis_name="c",
        subcore_axis_name="s",
        num_cores=NUM_CORES,
        num_subcores=NUM_SUBCORES,
    )
    out = pl.kernel(
        _embed_body,
        out_type=jax.ShapeDtypeStruct((B, S, HIDDEN), jnp.float32),
        mesh=mesh,
        scratch_types=[
            pltpu.VMEM((PER_W,), jnp.int32),
            pltpu.VMEM((PER_W,), jnp.int32),
            pltpu.VMEM((NG, CHUNK, HIDDEN), jnp.float32),
            pltpu.VMEM((NG, CHUNK, HIDDEN), jnp.float32),
            pltpu.VMEM((NO, CHUNK, HIDDEN), jnp.float32),
            pltpu.SemaphoreType.DMA((NG,)),
            pltpu.SemaphoreType.DMA((NG,)),
            pltpu.SemaphoreType.DMA((NO,)),
        ],
    )(ids, pids, word_emb, pos_emb)
    return out, attention_mask


# speedup vs baseline: 1.0192x; 1.0146x over previous
import jax
import jax.numpy as jnp
from jax import lax
from jax.experimental import pallas as pl
from jax.experimental.pallas import tpu as pltpu
from jax.experimental.pallas import tpu_sc as plsc

B, S, HIDDEN = 4, 2048, 1024
LANES = 16
NUM_CORES = 2
NUM_SUBCORES = 16
NW = NUM_CORES * NUM_SUBCORES
TOKENS = B * S
PER_W = TOKENS // NW
W_PER_ROW = S // PER_W
CHUNK = 16
NCHUNK = PER_W // CHUNK
CPH = HIDDEN // LANES
NS = 3


def _embed_body(ids, pids, wtab, ptab, out, idx_w, idx_p, buf_w, buf_p,
                sem_w, sem_p, sem_o):
    wid = lax.axis_index("s") * NUM_CORES + lax.axis_index("c")
    row = wid // W_PER_ROW
    col = (wid % W_PER_ROW) * PER_W
    pltpu.sync_copy(ids.at[row, pl.ds(col, PER_W)], idx_w)
    pltpu.sync_copy(pids.at[row, pl.ds(col, PER_W)], idx_p)

    def idx_slice(ref, j):
        return ref.at[pl.ds(pl.multiple_of(j * CHUNK, 8), CHUNK)]

    def gathers(j, s):
        cw = pltpu.make_async_copy(wtab.at[idx_slice(idx_w, j)],
                                   buf_w.at[s], sem_w.at[s])
        cp = pltpu.make_async_copy(ptab.at[idx_slice(idx_p, j)],
                                   buf_p.at[s], sem_p.at[s])
        return cw, cp

    def issue_gathers(j, s):
        cw, cp = gathers(j, s)
        cw.start()
        cp.start()

    def wait_gathers(j, s):
        cw, cp = gathers(j, s)
        cw.wait()
        cp.wait()

    def add_chunk(s):
        @plsc.parallel_loop(0, CHUNK * CPH, unroll=16)
        def _add(t):
            r = t >> 6
            c = pl.multiple_of((t & (CPH - 1)) << 4, LANES)
            plsc.addupdate(buf_w.at[s, r, pl.ds(c, LANES)],
                           buf_p[s, r, pl.ds(c, LANES)])

    def out_copy(j, s):
        tok = pl.multiple_of(col + j * CHUNK, 8)
        return pltpu.make_async_copy(buf_w.at[s], out.at[row, pl.ds(tok, CHUNK)],
                                     sem_o.at[s])

    issue_gathers(0, 0)
    issue_gathers(1, 1)
    wait_gathers(0, 0)
    add_chunk(0)
    issue_gathers(2, 2)
    out_copy(0, 0).start()

    def steady(j, carry):
        s = lax.rem(j, NS)
        sp = lax.rem(j - 1, NS)
        wait_gathers(j, s)
        out_copy(j - 1, sp).wait()
        add_chunk(s)
        issue_gathers(j + NS - 1, sp)
        out_copy(j, s).start()
        return carry

    lax.fori_loop(1, NCHUNK - NS + 1, steady, 0)

    for j in range(NCHUNK - NS + 1, NCHUNK):
        s = j % NS
        wait_gathers(j, s)
        add_chunk(s)
        out_copy(j, s).start()
    for j in range(NCHUNK - NS, NCHUNK):
        out_copy(j, j % NS).wait()


@jax.jit
def kernel(input_ids, position_ids, attention_mask, word_emb, pos_emb):
    ids = input_ids.astype(jnp.int32)
    pids = position_ids.astype(jnp.int32)
    mesh = plsc.VectorSubcoreMesh(
        core_axis_name="c",
        subcore_axis_name="s",
        num_cores=NUM_CORES,
        num_subcores=NUM_SUBCORES,
    )
    out = pl.kernel(
        _embed_body,
        out_type=jax.ShapeDtypeStruct((B, S, HIDDEN), jnp.float32),
        mesh=mesh,
        scratch_types=[
            pltpu.VMEM((PER_W,), jnp.int32),
            pltpu.VMEM((PER_W,), jnp.int32),
            pltpu.VMEM((NS, CHUNK, HIDDEN), jnp.float32),
            pltpu.VMEM((NS, CHUNK, HIDDEN), jnp.float32),
            pltpu.SemaphoreType.DMA((NS,)),
            pltpu.SemaphoreType.DMA((NS,)),
            pltpu.SemaphoreType.DMA((NS,)),
        ],
    )(ids, pids, word_emb, pos_emb)
    return out, attention_mask


# speedup vs baseline: 1.0223x; 1.0030x over previous
import jax
import jax.numpy as jnp
from jax import lax
from jax.experimental import pallas as pl
from jax.experimental.pallas import tpu as pltpu
from jax.experimental.pallas import tpu_sc as plsc

B, S, HIDDEN = 4, 2048, 1024
LANES = 16
NUM_CORES = 2
NUM_SUBCORES = 16
NW = NUM_CORES * NUM_SUBCORES
TOKENS = B * S
PER_W = TOKENS // NW
W_PER_ROW = S // PER_W
CHUNK = 16
NCHUNK = PER_W // CHUNK
CPH = HIDDEN // LANES
NS = 3


def _embed_body(ids, pids, wtab, ptab, out, idx_w, idx_p, buf_w, buf_p,
                sem_w, sem_p, sem_o):
    wid = lax.axis_index("s") * NUM_CORES + lax.axis_index("c")
    row = wid // W_PER_ROW
    col = (wid % W_PER_ROW) * PER_W
    pltpu.sync_copy(ids.at[row, pl.ds(col, PER_W)], idx_w)
    pltpu.sync_copy(pids.at[row, pl.ds(col, PER_W)], idx_p)

    def idx_slice(ref, j):
        return ref.at[pl.ds(pl.multiple_of(j * CHUNK, 8), CHUNK)]

    def gathers(j, s):
        cw = pltpu.make_async_copy(wtab.at[idx_slice(idx_w, j)],
                                   buf_w.at[s], sem_w.at[s])
        cp = pltpu.make_async_copy(ptab.at[idx_slice(idx_p, j)],
                                   buf_p.at[s], sem_p.at[s])
        return cw, cp

    def issue_gathers(j, s):
        cw, cp = gathers(j, s)
        cw.start()
        cp.start()

    def wait_gathers(j, s):
        cw, cp = gathers(j, s)
        cw.wait()
        cp.wait()

    def add_chunk(s):
        @plsc.parallel_loop(0, CHUNK * CPH, unroll=16)
        def _add(t):
            r = t >> 6
            c = pl.multiple_of((t & (CPH - 1)) << 4, LANES)
            plsc.addupdate(buf_w.at[s, r, pl.ds(c, LANES)],
                           buf_p[s, r, pl.ds(c, LANES)])

    def out_copy(j, s):
        tok = pl.multiple_of(col + j * CHUNK, 8)
        return pltpu.make_async_copy(buf_w.at[s], out.at[row, pl.ds(tok, CHUNK)],
                                     sem_o.at[s])

    for j in range(NS):
        issue_gathers(j, j)
    wait_gathers(0, 0)
    add_chunk(0)
    out_copy(0, 0).start()

    def steady(j, carry):
        s = lax.rem(j, NS)
        sp = lax.rem(j - 1, NS)
        wait_gathers(j, s)
        out_copy(j - 1, sp).wait()
        add_chunk(s)
        issue_gathers(j + NS - 1, sp)
        out_copy(j, s).start()
        return carry

    lax.fori_loop(1, NCHUNK - NS + 1, steady, 0)

    for j in range(NCHUNK - NS + 1, NCHUNK):
        s = j % NS
        wait_gathers(j, s)
        add_chunk(s)
        out_copy(j, s).start()
    for j in range(NCHUNK - NS, NCHUNK):
        out_copy(j, j % NS).wait()


@jax.jit
def kernel(input_ids, position_ids, attention_mask, word_emb, pos_emb):
    ids = input_ids.astype(jnp.int32)
    pids = position_ids.astype(jnp.int32)
    mesh = plsc.VectorSubcoreMesh(
        core_axis_name="c",
        subcore_axis_name="s",
        num_cores=NUM_CORES,
        num_subcores=NUM_SUBCORES,
    )
    out = pl.kernel(
        _embed_body,
        out_type=jax.ShapeDtypeStruct((B, S, HIDDEN), jnp.float32),
        mesh=mesh,
        scratch_types=[
            pltpu.VMEM((PER_W,), jnp.int32),
            pltpu.VMEM((PER_W,), jnp.int32),
            pltpu.VMEM((NS, CHUNK, HIDDEN), jnp.float32),
            pltpu.VMEM((NS, CHUNK, HIDDEN), jnp.float32),
            pltpu.SemaphoreType.DMA((NS,)),
            pltpu.SemaphoreType.DMA((NS,)),
            pltpu.SemaphoreType.DMA((NS,)),
        ],
    )(ids, pids, word_emb, pos_emb)
    return out, attention_mask
